# baseline (speedup 1.0000x reference)
import functools

import jax
import jax.numpy as jnp
from jax import lax
from jax.experimental import pallas as pl
from jax.experimental.pallas import tpu as pltpu
from jax.experimental.pallas import tpu_sc as plsc

_EMBED = 32
_BATCH = 16384
_NC = 2
_NS = 16
_NW = _NC * _NS
_BPW = _BATCH // _NW


def _gather_embeddings(user_table, genre_table, user_idx, genre_idx):
    mesh = plsc.VectorSubcoreMesh(core_axis_name="c", subcore_axis_name="s")

    @functools.partial(
        pl.kernel,
        mesh=mesh,
        compiler_params=pltpu.CompilerParams(use_tc_tiling_on_sc=False),
        out_type=[
            jax.ShapeDtypeStruct((_BATCH, _EMBED), jnp.float32),
            jax.ShapeDtypeStruct((_BATCH, _EMBED), jnp.float32),
        ],
        scratch_types=[
            pltpu.VMEM((_BPW,), jnp.int32),
            pltpu.VMEM((_BPW, _EMBED), jnp.float32),
            pltpu.VMEM((_BPW,), jnp.int32),
            pltpu.VMEM((_BPW, _EMBED), jnp.float32),
            pltpu.SemaphoreType.DMA,
            pltpu.SemaphoreType.DMA,
        ],
    )
    def k(ut_hbm, gt_hbm, uidx_hbm, gidx_hbm, uout_hbm, gout_hbm,
          uidx_v, urows_v, gidx_v, grows_v, usem, gsem):
        wid = lax.axis_index("s") * _NC + lax.axis_index("c")
        base = wid * _BPW
        pltpu.sync_copy(uidx_hbm.at[pl.ds(base, _BPW)], uidx_v)
        pltpu.sync_copy(gidx_hbm.at[pl.ds(base, _BPW)], gidx_v)
        ucp = pltpu.async_copy(ut_hbm.at[uidx_v], urows_v, usem)
        gcp = pltpu.async_copy(gt_hbm.at[gidx_v], grows_v, gsem)
        ucp.wait()
        pltpu.sync_copy(urows_v, uout_hbm.at[pl.ds(base, _BPW)])
        gcp.wait()
        pltpu.sync_copy(grows_v, gout_hbm.at[pl.ds(base, _BPW)])

    return k(user_table, genre_table, user_idx, genre_idx)


def _mlp_body(u_ref, g_ref, w1u_ref, w1g_ref, b1_ref, w2_ref, b2_ref,
              w3_ref, b3_ref, o_ref):
    h = jnp.dot(u_ref[...], w1u_ref[...], preferred_element_type=jnp.float32)
    h += jnp.dot(g_ref[...], w1g_ref[...], preferred_element_type=jnp.float32)
    h = jnp.maximum(h + b1_ref[...], 0.0)
    h = jnp.maximum(
        jnp.dot(h, w2_ref[...], preferred_element_type=jnp.float32) + b2_ref[...],
        0.0)
    o_ref[...] = (
        jnp.dot(h, w3_ref[...], preferred_element_type=jnp.float32) + b3_ref[...])


def _mlp(u, g, W1u, W1g, b1, W2, b2, W3, b3):
    bm = 2048
    h1 = W1u.shape[1]
    h2 = W2.shape[1]
    h3 = W3.shape[1]
    return pl.pallas_call(
        _mlp_body,
        grid=(_BATCH // bm,),
        in_specs=[
            pl.BlockSpec((bm, _EMBED), lambda i: (i, 0)),
            pl.BlockSpec((bm, _EMBED), lambda i: (i, 0)),
            pl.BlockSpec((_EMBED, h1), lambda i: (0, 0)),
            pl.BlockSpec((_EMBED, h1), lambda i: (0, 0)),
            pl.BlockSpec((1, h1), lambda i: (0, 0)),
            pl.BlockSpec((h1, h2), lambda i: (0, 0)),
            pl.BlockSpec((1, h2), lambda i: (0, 0)),
            pl.BlockSpec((h2, h3), lambda i: (0, 0)),
            pl.BlockSpec((1, h3), lambda i: (0, 0)),
        ],
        out_specs=pl.BlockSpec((bm, h3), lambda i: (i, 0)),
        out_shape=jax.ShapeDtypeStruct((_BATCH, h3), jnp.float32),
    )(u, g, W1u, W1g, b1.reshape(1, -1), W2, b2.reshape(1, -1), W3,
      b3.reshape(1, -1))


def kernel(inputs, user_table, genre_table, W1, b1, W2, b2, W3, b3):
    user_idx = inputs[:, 0]
    genre_idx = inputs[:, 1]
    u, g = _gather_embeddings(user_table, genre_table, user_idx, genre_idx)
    W1u = W1[:_EMBED]
    W1g = W1[_EMBED:]
    return _mlp(u, g, W1u, W1g, b1, W2, b2, W3, b3)
